# baseline (speedup 1.0000x reference)
import functools

import jax
import jax.numpy as jnp
from jax import lax
from jax.experimental import pallas as pl
from jax.experimental.pallas import tpu as pltpu
from jax.experimental.pallas import tpu_sc as plsc

_NC = 2
_NS = 16
_NW = _NC * _NS
_CW = 16
_CH = 400
_HD = 64


def _dot_t(a, w):
    return lax.dot_general(a, w, (((1,), (1,)), ((), ())),
                           preferred_element_type=jnp.float32)


def _tc_pre(x, wl, wr, b):
    N, D = x.shape
    BLK = 1000
    def body(x_ref, wl_ref, wr_ref, b_ref, xl_ref, xr_ref):
        xb = x_ref[...]
        xl_ref[...] = _dot_t(xb, wl_ref[...])
        xr_ref[...] = _dot_t(xb, wr_ref[...]) + b_ref[...]
    return pl.pallas_call(
        body,
        grid=(N // BLK,),
        in_specs=[pl.BlockSpec((BLK, D), lambda i: (i, 0)),
                  pl.BlockSpec((D, D), lambda i: (0, 0)),
                  pl.BlockSpec((D, D), lambda i: (0, 0)),
                  pl.BlockSpec((1, D), lambda i: (0, 0))],
        out_specs=[pl.BlockSpec((BLK, D), lambda i: (i, 0)),
                   pl.BlockSpec((BLK, D), lambda i: (i, 0))],
        out_shape=[jax.ShapeDtypeStruct((N, D), jnp.float32),
                   jax.ShapeDtypeStruct((N, D), jnp.float32)],
    )(x, wl, wr, b.reshape(1, D))


def _tc_mid(p, c, xr, wl, wr, b):
    N, D = xr.shape
    BLK = 1000
    def body(p_ref, c_ref, xr_ref, wl_ref, wr_ref, b_ref,
             hl_ref, hr_ref):
        s = p_ref[0] + p_ref[1]
        cnt = c_ref[0, :, 0:1] + c_ref[1, :, 0:1]
        h = jnp.maximum(s / jnp.maximum(cnt, 1.0) + xr_ref[...], 0.0)
        hl_ref[...] = _dot_t(h, wl_ref[...])
        hr_ref[...] = _dot_t(h, wr_ref[...]) + b_ref[...]
    return pl.pallas_call(
        body,
        grid=(N // BLK,),
        in_specs=[pl.BlockSpec((_NC, BLK, D), lambda i: (0, i, 0)),
                  pl.BlockSpec((_NC, BLK, _CW), lambda i: (0, i, 0)),
                  pl.BlockSpec((BLK, D), lambda i: (i, 0)),
                  pl.BlockSpec((D, D), lambda i: (0, 0)),
                  pl.BlockSpec((D, D), lambda i: (0, 0)),
                  pl.BlockSpec((1, D), lambda i: (0, 0))],
        out_specs=[pl.BlockSpec((BLK, D), lambda i: (i, 0)),
                   pl.BlockSpec((BLK, D), lambda i: (i, 0))],
        out_shape=[jax.ShapeDtypeStruct((N, D), jnp.float32),
                   jax.ShapeDtypeStruct((N, D), jnp.float32)],
    )(p, c, xr, wl, wr, b.reshape(1, D))


def _tc_post(p, c, hr):
    N, D = hr.shape
    BLK = 1000
    def body(p_ref, c_ref, hr_ref, o_ref):
        s = p_ref[0] + p_ref[1]
        cnt = c_ref[0, :, 0:1] + c_ref[1, :, 0:1]
        o_ref[...] = s / jnp.maximum(cnt, 1.0) + hr_ref[...]
    return pl.pallas_call(
        body,
        grid=(N // BLK,),
        in_specs=[pl.BlockSpec((_NC, BLK, D), lambda i: (0, i, 0)),
                  pl.BlockSpec((_NC, BLK, _CW), lambda i: (0, i, 0)),
                  pl.BlockSpec((BLK, D), lambda i: (i, 0))],
        out_specs=pl.BlockSpec((BLK, D), lambda i: (i, 0)),
        out_shape=jax.ShapeDtypeStruct((N, D), jnp.float32),
    )(p, c, hr)


def _sc_agg(y2, src2_i, dst_i, zeros_nd):
    NP = zeros_nd.shape[0] * _NS
    _, NCH, CH = src2_i.shape
    RPT = NP // _NS
    mesh = plsc.VectorSubcoreMesh(core_axis_name="c", subcore_axis_name="s")
    NE = NCH if NCH % 2 == 0 else NCH - 1

    out_type = jax.ShapeDtypeStruct((_NC, NP, 2 * _HD), jnp.float32)
    scratch = [pltpu.VMEM((NCH, CH), jnp.int32),
               pltpu.VMEM((NCH, CH), jnp.int32),
               pltpu.VMEM((CH, _HD), jnp.float32),
               pltpu.VMEM((CH, _HD), jnp.float32),
               pltpu.SemaphoreType.DMA,
               pltpu.SemaphoreType.DMA,
               pltpu.VMEM_SHARED((NP, _HD), jnp.float32)]

    @functools.partial(pl.kernel, out_type=out_type, mesh=mesh,
                       scratch_types=scratch,
                       compiler_params=pltpu.CompilerParams(
                           use_tc_tiling_on_sc=False))
    def k(y_h, src_h, dst_h, znd_h, out_h,
          srcv, dstv, rows0, rows1, sem0, sem1, acc):
        cid = lax.axis_index("c")
        sid = lax.axis_index("s")
        wid = cid * _NS + sid
        r0 = sid * RPT
        rs = pl.ds(r0, RPT)
        pltpu.sync_copy(src_h.at[wid], srcv)
        pltpu.sync_copy(dst_h.at[wid], dstv)
        for half in (0, 1):
            if half == 1:
                @pl.loop(0, NCH)
                def _(j):
                    @pl.loop(0, CH, step=16)
                    def _(k, j=j):
                        sl = (pl.ds(j, 1), pl.ds(k, 16))
                        srcv.at[sl][...] = srcv.at[sl][...] + 1
            pltpu.sync_copy(znd_h, acc.at[rs])
            plsc.subcore_barrier()
            pltpu.make_async_copy(y_h.at[srcv.at[0]], rows0, sem0).start()
            pltpu.make_async_copy(y_h.at[srcv.at[1]], rows1, sem1).start()

            @pl.loop(0, NE, step=2)
            def _(j):
                pltpu.make_async_copy(y_h.at[srcv.at[j]], rows0, sem0).wait()
                pltpu.sync_copy(rows0, acc.at[dstv.at[j]], add=True)

                @pl.when(j + 2 < NCH)
                def _():
                    pltpu.make_async_copy(
                        y_h.at[srcv.at[j + 2]], rows0, sem0).start()

                pltpu.make_async_copy(
                    y_h.at[srcv.at[j + 1]], rows1, sem1).wait()
                pltpu.sync_copy(rows1, acc.at[dstv.at[j + 1]], add=True)

                @pl.when(j + 3 < NCH)
                def _():
                    pltpu.make_async_copy(
                        y_h.at[srcv.at[j + 3]], rows1, sem1).start()

            if NCH % 2:
                j = NCH - 1
                pltpu.make_async_copy(y_h.at[srcv.at[j]], rows0, sem0).wait()
                pltpu.sync_copy(rows0, acc.at[dstv.at[j]], add=True)

            plsc.subcore_barrier()
            pltpu.sync_copy(acc.at[rs],
                            out_h.at[cid, rs, pl.ds(half * _HD, _HD)])
            plsc.subcore_barrier()

    return k(y2, src2_i, dst_i, zeros_nd)


def _sc_count(dst_i, zeros_nc, ones_c):
    NP = zeros_nc.shape[0] * _NS
    _, NCH, CH = dst_i.shape
    RPT = NP // _NS
    mesh = plsc.VectorSubcoreMesh(core_axis_name="c", subcore_axis_name="s")

    @functools.partial(
        pl.kernel,
        out_type=jax.ShapeDtypeStruct((_NC, NP, _CW), jnp.float32),
        mesh=mesh,
        scratch_types=[pltpu.VMEM((NCH, CH), jnp.int32),
                       pltpu.VMEM((CH, _CW), jnp.float32),
                       pltpu.VMEM_SHARED((NP, _CW), jnp.float32)],
        compiler_params=pltpu.CompilerParams(use_tc_tiling_on_sc=False))
    def k(dst_h, znc_h, ones_h, outc_h, dstv, ones_v, accc):
        cid = lax.axis_index("c")
        sid = lax.axis_index("s")
        wid = cid * _NS + sid
        r0 = sid * RPT
        rs = pl.ds(r0, RPT)
        pltpu.sync_copy(dst_h.at[wid], dstv)
        pltpu.sync_copy(ones_h, ones_v)
        pltpu.sync_copy(znc_h, accc.at[rs])
        plsc.subcore_barrier()

        @pl.loop(0, NCH)
        def _(j):
            pltpu.sync_copy(ones_v, accc.at[dstv.at[j]], add=True)

        plsc.subcore_barrier()
        pltpu.sync_copy(accc.at[rs], outc_h.at[cid, rs])

    return k(dst_i, zeros_nc, ones_c)


def kernel(x, edge_index, W1_l, b1_l, W1_r, W2_l, b2_l, W2_r):
    N, D = x.shape
    E = edge_index.shape[1]
    ei = edge_index.astype(jnp.int32)
    nch = E // (_NW * _CH)
    src2_i = (ei[0] * 2).reshape(_NW, nch, _CH)
    dst_i = ei[1].reshape(_NW, nch, _CH)
    npad = -(-N // (8 * _NS)) * (8 * _NS)
    zeros_nd = jnp.zeros((npad // _NS, _HD), jnp.float32)
    zeros_nc = jnp.zeros((npad // _NS, _CW), jnp.float32)
    ones_c = jnp.ones((_CH, _CW), jnp.float32)

    c1 = _sc_count(dst_i, zeros_nc, ones_c)
    xl, xr = _tc_pre(x, W1_l, W1_r, b1_l)
    p1 = _sc_agg(xl.reshape(2 * N, _HD), src2_i, dst_i, zeros_nd)
    hl, hr = _tc_mid(p1, c1, xr, W2_l, W2_r, b2_l)
    p2 = _sc_agg(hl.reshape(2 * N, _HD), src2_i, dst_i, zeros_nd)
    return _tc_post(p2, c1, hr)


# speedup vs baseline: 1.0142x; 1.0142x over previous
import functools

import jax
import jax.numpy as jnp
from jax import lax
from jax.experimental import pallas as pl
from jax.experimental.pallas import tpu as pltpu
from jax.experimental.pallas import tpu_sc as plsc

_NC = 2
_NS = 16
_NW = _NC * _NS
_CW = 16
_CH = 400
_HD = 64


def _dot_t(a, w):
    return lax.dot_general(a, w, (((1,), (1,)), ((), ())),
                           preferred_element_type=jnp.float32)


def _tc_pre(x, wl, wr, b):
    N, D = x.shape
    BLK = 1000
    def body(x_ref, wl_ref, wr_ref, b_ref, xl_ref, xr_ref):
        xb = x_ref[...]
        xl_ref[...] = _dot_t(xb, wl_ref[...])
        xr_ref[...] = _dot_t(xb, wr_ref[...]) + b_ref[...]
    return pl.pallas_call(
        body,
        grid=(N // BLK,),
        in_specs=[pl.BlockSpec((BLK, D), lambda i: (i, 0)),
                  pl.BlockSpec((D, D), lambda i: (0, 0)),
                  pl.BlockSpec((D, D), lambda i: (0, 0)),
                  pl.BlockSpec((1, D), lambda i: (0, 0))],
        out_specs=[pl.BlockSpec((BLK, D), lambda i: (i, 0)),
                   pl.BlockSpec((BLK, D), lambda i: (i, 0))],
        out_shape=[jax.ShapeDtypeStruct((N, D), jnp.float32),
                   jax.ShapeDtypeStruct((N, D), jnp.float32)],
    )(x, wl, wr, b.reshape(1, D))


def _tc_mid(p, c, xr, wl, wr, b):
    N, D = xr.shape
    BLK = 1000
    def body(p_ref, c_ref, xr_ref, wl_ref, wr_ref, b_ref,
             hl_ref, hr_ref):
        s = p_ref[0] + p_ref[1]
        cnt = c_ref[0, :, 0:1] + c_ref[1, :, 0:1]
        h = jnp.maximum(s / jnp.maximum(cnt, 1.0) + xr_ref[...], 0.0)
        hl_ref[...] = _dot_t(h, wl_ref[...])
        hr_ref[...] = _dot_t(h, wr_ref[...]) + b_ref[...]
    return pl.pallas_call(
        body,
        grid=(N // BLK,),
        in_specs=[pl.BlockSpec((_NC, BLK, D), lambda i: (0, i, 0)),
                  pl.BlockSpec((_NC, BLK, _CW), lambda i: (0, i, 0)),
                  pl.BlockSpec((BLK, D), lambda i: (i, 0)),
                  pl.BlockSpec((D, D), lambda i: (0, 0)),
                  pl.BlockSpec((D, D), lambda i: (0, 0)),
                  pl.BlockSpec((1, D), lambda i: (0, 0))],
        out_specs=[pl.BlockSpec((BLK, D), lambda i: (i, 0)),
                   pl.BlockSpec((BLK, D), lambda i: (i, 0))],
        out_shape=[jax.ShapeDtypeStruct((N, D), jnp.float32),
                   jax.ShapeDtypeStruct((N, D), jnp.float32)],
    )(p, c, xr, wl, wr, b.reshape(1, D))


def _tc_post(p, c, hr):
    N, D = hr.shape
    BLK = 1000
    def body(p_ref, c_ref, hr_ref, o_ref):
        s = p_ref[0] + p_ref[1]
        cnt = c_ref[0, :, 0:1] + c_ref[1, :, 0:1]
        o_ref[...] = s / jnp.maximum(cnt, 1.0) + hr_ref[...]
    return pl.pallas_call(
        body,
        grid=(N // BLK,),
        in_specs=[pl.BlockSpec((_NC, BLK, D), lambda i: (0, i, 0)),
                  pl.BlockSpec((_NC, BLK, _CW), lambda i: (0, i, 0)),
                  pl.BlockSpec((BLK, D), lambda i: (i, 0))],
        out_specs=pl.BlockSpec((BLK, D), lambda i: (i, 0)),
        out_shape=jax.ShapeDtypeStruct((N, D), jnp.float32),
    )(p, c, hr)


def _sc_agg(y2, src2_i, dst_i, zeros_nd):
    NP = zeros_nd.shape[0]
    _, NCH, CH = src2_i.shape
    RPT = NP // _NS
    mesh = plsc.VectorSubcoreMesh(core_axis_name="c", subcore_axis_name="s")
    NE = NCH if NCH % 2 == 0 else NCH - 1

    out_type = jax.ShapeDtypeStruct((_NC, NP, 2 * _HD), jnp.float32)
    scratch = [pltpu.VMEM((NCH, CH), jnp.int32),
               pltpu.VMEM((NCH, CH), jnp.int32),
               pltpu.VMEM((CH, _HD), jnp.float32),
               pltpu.VMEM((CH, _HD), jnp.float32),
               pltpu.SemaphoreType.DMA,
               pltpu.SemaphoreType.DMA,
               pltpu.VMEM_SHARED((NP, _HD), jnp.float32)]

    @functools.partial(pl.kernel, out_type=out_type, mesh=mesh,
                       scratch_types=scratch,
                       compiler_params=pltpu.CompilerParams(
                           use_tc_tiling_on_sc=False))
    def k(y_h, src_h, dst_h, znd_h, out_h,
          srcv, dstv, rows0, rows1, sem0, sem1, acc):
        cid = lax.axis_index("c")
        sid = lax.axis_index("s")
        wid = cid * _NS + sid
        r0 = sid * RPT
        rs = pl.ds(r0, RPT)
        pltpu.sync_copy(src_h.at[wid], srcv)
        pltpu.sync_copy(dst_h.at[wid], dstv)
        for half in (0, 1):
            if half == 1:
                @pl.loop(0, NCH)
                def _(j):
                    @pl.loop(0, CH, step=16)
                    def _(k, j=j):
                        sl = (pl.ds(j, 1), pl.ds(k, 16))
                        srcv.at[sl][...] = srcv.at[sl][...] + 1
            pltpu.sync_copy(znd_h.at[rs], acc.at[rs])
            plsc.subcore_barrier()
            pltpu.make_async_copy(y_h.at[srcv.at[0]], rows0, sem0).start()
            pltpu.make_async_copy(y_h.at[srcv.at[1]], rows1, sem1).start()

            @pl.loop(0, NE, step=2)
            def _(j):
                pltpu.make_async_copy(y_h.at[srcv.at[j]], rows0, sem0).wait()
                pltpu.sync_copy(rows0, acc.at[dstv.at[j]], add=True)

                @pl.when(j + 2 < NCH)
                def _():
                    pltpu.make_async_copy(
                        y_h.at[srcv.at[j + 2]], rows0, sem0).start()

                pltpu.make_async_copy(
                    y_h.at[srcv.at[j + 1]], rows1, sem1).wait()
                pltpu.sync_copy(rows1, acc.at[dstv.at[j + 1]], add=True)

                @pl.when(j + 3 < NCH)
                def _():
                    pltpu.make_async_copy(
                        y_h.at[srcv.at[j + 3]], rows1, sem1).start()

            if NCH % 2:
                j = NCH - 1
                pltpu.make_async_copy(y_h.at[srcv.at[j]], rows0, sem0).wait()
                pltpu.sync_copy(rows0, acc.at[dstv.at[j]], add=True)

            plsc.subcore_barrier()
            pltpu.sync_copy(acc.at[rs],
                            out_h.at[cid, rs, pl.ds(half * _HD, _HD)])
            plsc.subcore_barrier()

    return k(y2, src2_i, dst_i, zeros_nd)


def _sc_count(dst_i, zeros_nc, ones_c):
    NP = zeros_nc.shape[0]
    _, NCH, CH = dst_i.shape
    RPT = NP // _NS
    mesh = plsc.VectorSubcoreMesh(core_axis_name="c", subcore_axis_name="s")

    @functools.partial(
        pl.kernel,
        out_type=jax.ShapeDtypeStruct((_NC, NP, _CW), jnp.float32),
        mesh=mesh,
        scratch_types=[pltpu.VMEM((NCH, CH), jnp.int32),
                       pltpu.VMEM((CH, _CW), jnp.float32),
                       pltpu.VMEM_SHARED((NP, _CW), jnp.float32)],
        compiler_params=pltpu.CompilerParams(use_tc_tiling_on_sc=False))
    def k(dst_h, znc_h, ones_h, outc_h, dstv, ones_v, accc):
        cid = lax.axis_index("c")
        sid = lax.axis_index("s")
        wid = cid * _NS + sid
        r0 = sid * RPT
        rs = pl.ds(r0, RPT)
        pltpu.sync_copy(dst_h.at[wid], dstv)
        pltpu.sync_copy(ones_h, ones_v)
        pltpu.sync_copy(znc_h.at[rs], accc.at[rs])
        plsc.subcore_barrier()

        @pl.loop(0, NCH)
        def _(j):
            pltpu.sync_copy(ones_v, accc.at[dstv.at[j]], add=True)

        plsc.subcore_barrier()
        pltpu.sync_copy(accc.at[rs], outc_h.at[cid, rs])

    return k(dst_i, zeros_nc, ones_c)


def kernel(x, edge_index, W1_l, b1_l, W1_r, W2_l, b2_l, W2_r):
    N, D = x.shape
    E = edge_index.shape[1]
    ei = edge_index.astype(jnp.int32)
    nch = E // (_NW * _CH)
    src2_i = (ei[0] * 2).reshape(_NW, nch, _CH)
    dst_i = ei[1].reshape(_NW, nch, _CH)
    npad = -(-N // (8 * _NS)) * (8 * _NS)
    zeros_nd = jnp.zeros((npad, _HD), jnp.float32)
    zeros_nc = jnp.zeros((npad, _CW), jnp.float32)
    ones_c = jnp.ones((_CH, _CW), jnp.float32)

    c1 = _sc_count(dst_i, zeros_nc, ones_c)
    xl, xr = _tc_pre(x, W1_l, W1_r, b1_l)
    p1 = _sc_agg(xl.reshape(2 * N, _HD), src2_i, dst_i, zeros_nd)
    hl, hr = _tc_mid(p1, c1, xr, W2_l, W2_r, b2_l)
    p2 = _sc_agg(hl.reshape(2 * N, _HD), src2_i, dst_i, zeros_nd)
    return _tc_post(p2, c1, hr)


# speedup vs baseline: 1.0387x; 1.0241x over previous
import functools

import jax
import jax.numpy as jnp
from jax import lax
from jax.experimental import pallas as pl
from jax.experimental.pallas import tpu as pltpu
from jax.experimental.pallas import tpu_sc as plsc

_NC = 2
_NS = 16
_NW = _NC * _NS
_CW = 16
_CH = 400
_HD = 64


def _dot_t(a, w):
    return lax.dot_general(a, w, (((1,), (1,)), ((), ())),
                           preferred_element_type=jnp.float32)


def _tc_pre(x, wl, wr, b):
    N, D = x.shape
    BLK = 1000
    def body(x_ref, wl_ref, wr_ref, b_ref, xl_ref, xr_ref):
        xb = x_ref[...]
        xl_ref[...] = _dot_t(xb, wl_ref[...])
        xr_ref[...] = _dot_t(xb, wr_ref[...]) + b_ref[...]
    return pl.pallas_call(
        body,
        grid=(N // BLK,),
        in_specs=[pl.BlockSpec((BLK, D), lambda i: (i, 0)),
                  pl.BlockSpec((D, D), lambda i: (0, 0)),
                  pl.BlockSpec((D, D), lambda i: (0, 0)),
                  pl.BlockSpec((1, D), lambda i: (0, 0))],
        out_specs=[pl.BlockSpec((BLK, D), lambda i: (i, 0)),
                   pl.BlockSpec((BLK, D), lambda i: (i, 0))],
        out_shape=[jax.ShapeDtypeStruct((N, D), jnp.float32),
                   jax.ShapeDtypeStruct((N, D), jnp.float32)],
    )(x, wl, wr, b.reshape(1, D))


def _tc_mid(p, c, xr, wl, wr, b):
    N, D = xr.shape
    BLK = 1000
    def body(p_ref, c_ref, xr_ref, wl_ref, wr_ref, b_ref,
             hl_ref, hr_ref):
        s = p_ref[0] + p_ref[1]
        cnt = c_ref[0, :, 0:1] + c_ref[1, :, 0:1]
        h = jnp.maximum(s / jnp.maximum(cnt, 1.0) + xr_ref[...], 0.0)
        hl_ref[...] = _dot_t(h, wl_ref[...])
        hr_ref[...] = _dot_t(h, wr_ref[...]) + b_ref[...]
    return pl.pallas_call(
        body,
        grid=(N // BLK,),
        in_specs=[pl.BlockSpec((_NC, BLK, D), lambda i: (0, i, 0)),
                  pl.BlockSpec((_NC, BLK, _CW), lambda i: (0, i, 0)),
                  pl.BlockSpec((BLK, D), lambda i: (i, 0)),
                  pl.BlockSpec((D, D), lambda i: (0, 0)),
                  pl.BlockSpec((D, D), lambda i: (0, 0)),
                  pl.BlockSpec((1, D), lambda i: (0, 0))],
        out_specs=[pl.BlockSpec((BLK, D), lambda i: (i, 0)),
                   pl.BlockSpec((BLK, D), lambda i: (i, 0))],
        out_shape=[jax.ShapeDtypeStruct((N, D), jnp.float32),
                   jax.ShapeDtypeStruct((N, D), jnp.float32)],
    )(p, c, xr, wl, wr, b.reshape(1, D))


def _tc_post(p, c, hr):
    N, D = hr.shape
    BLK = 1000
    def body(p_ref, c_ref, hr_ref, o_ref):
        s = p_ref[0] + p_ref[1]
        cnt = c_ref[0, :, 0:1] + c_ref[1, :, 0:1]
        o_ref[...] = s / jnp.maximum(cnt, 1.0) + hr_ref[...]
    return pl.pallas_call(
        body,
        grid=(N // BLK,),
        in_specs=[pl.BlockSpec((_NC, BLK, D), lambda i: (0, i, 0)),
                  pl.BlockSpec((_NC, BLK, _CW), lambda i: (0, i, 0)),
                  pl.BlockSpec((BLK, D), lambda i: (i, 0))],
        out_specs=pl.BlockSpec((BLK, D), lambda i: (i, 0)),
        out_shape=jax.ShapeDtypeStruct((N, D), jnp.float32),
    )(p, c, hr)


def _sc_agg(y2, src2_i, src2b_i, dst_i, zeros_nd):
    NP = zeros_nd.shape[0]
    _, NCH, CH = src2_i.shape
    RPT = NP // _NS
    mesh = plsc.VectorSubcoreMesh(core_axis_name="c", subcore_axis_name="s")
    NE = NCH if NCH % 2 == 0 else NCH - 1

    out_type = jax.ShapeDtypeStruct((_NC, NP, 2 * _HD), jnp.float32)
    scratch = [pltpu.VMEM((NCH, CH), jnp.int32),
               pltpu.VMEM((NCH, CH), jnp.int32),
               pltpu.VMEM((NCH, CH), jnp.int32),
               pltpu.VMEM((CH, _HD), jnp.float32),
               pltpu.VMEM((CH, _HD), jnp.float32),
               pltpu.SemaphoreType.DMA,
               pltpu.SemaphoreType.DMA,
               pltpu.VMEM_SHARED((NP, _HD), jnp.float32)]

    @functools.partial(pl.kernel, out_type=out_type, mesh=mesh,
                       scratch_types=scratch,
                       compiler_params=pltpu.CompilerParams(
                           use_tc_tiling_on_sc=False))
    def k(y_h, src_h, srcb_h, dst_h, znd_h, out_h,
          srcv0, srcv1, dstv, rows0, rows1, sem0, sem1, acc):
        cid = lax.axis_index("c")
        sid = lax.axis_index("s")
        wid = cid * _NS + sid
        r0 = sid * RPT
        rs = pl.ds(r0, RPT)
        pltpu.sync_copy(src_h.at[wid], srcv0)
        pltpu.sync_copy(srcb_h.at[wid], srcv1)
        pltpu.sync_copy(dst_h.at[wid], dstv)
        for half, srcv in ((0, srcv0), (1, srcv1)):
            pltpu.make_async_copy(y_h.at[srcv.at[0]], rows0, sem0).start()
            pltpu.make_async_copy(y_h.at[srcv.at[1]], rows1, sem1).start()
            pltpu.sync_copy(znd_h.at[rs], acc.at[rs])
            plsc.subcore_barrier()

            @pl.loop(0, NE, step=2)
            def _(j, srcv=srcv):
                pltpu.make_async_copy(y_h.at[srcv.at[j]], rows0, sem0).wait()
                pltpu.sync_copy(rows0, acc.at[dstv.at[j]], add=True)

                @pl.when(j + 2 < NCH)
                def _():
                    pltpu.make_async_copy(
                        y_h.at[srcv.at[j + 2]], rows0, sem0).start()

                pltpu.make_async_copy(
                    y_h.at[srcv.at[j + 1]], rows1, sem1).wait()
                pltpu.sync_copy(rows1, acc.at[dstv.at[j + 1]], add=True)

                @pl.when(j + 3 < NCH)
                def _():
                    pltpu.make_async_copy(
                        y_h.at[srcv.at[j + 3]], rows1, sem1).start()

            if NCH % 2:
                j = NCH - 1
                pltpu.make_async_copy(y_h.at[srcv.at[j]], rows0, sem0).wait()
                pltpu.sync_copy(rows0, acc.at[dstv.at[j]], add=True)

            plsc.subcore_barrier()
            pltpu.sync_copy(acc.at[rs],
                            out_h.at[cid, rs, pl.ds(half * _HD, _HD)])

    return k(y2, src2_i, src2b_i, dst_i, zeros_nd)


def _sc_count(dst_i, zeros_nc, ones_c):
    NP = zeros_nc.shape[0]
    _, NCH, CH = dst_i.shape
    RPT = NP // _NS
    mesh = plsc.VectorSubcoreMesh(core_axis_name="c", subcore_axis_name="s")

    @functools.partial(
        pl.kernel,
        out_type=jax.ShapeDtypeStruct((_NC, NP, _CW), jnp.float32),
        mesh=mesh,
        scratch_types=[pltpu.VMEM((NCH, CH), jnp.int32),
                       pltpu.VMEM((CH, _CW), jnp.float32),
                       pltpu.VMEM_SHARED((NP, _CW), jnp.float32)],
        compiler_params=pltpu.CompilerParams(use_tc_tiling_on_sc=False))
    def k(dst_h, znc_h, ones_h, outc_h, dstv, ones_v, accc):
        cid = lax.axis_index("c")
        sid = lax.axis_index("s")
        wid = cid * _NS + sid
        r0 = sid * RPT
        rs = pl.ds(r0, RPT)
        pltpu.sync_copy(dst_h.at[wid], dstv)
        pltpu.sync_copy(ones_h, ones_v)
        pltpu.sync_copy(znc_h.at[rs], accc.at[rs])
        plsc.subcore_barrier()

        @pl.loop(0, NCH)
        def _(j):
            pltpu.sync_copy(ones_v, accc.at[dstv.at[j]], add=True)

        plsc.subcore_barrier()
        pltpu.sync_copy(accc.at[rs], outc_h.at[cid, rs])

    return k(dst_i, zeros_nc, ones_c)


def kernel(x, edge_index, W1_l, b1_l, W1_r, W2_l, b2_l, W2_r):
    N, D = x.shape
    E = edge_index.shape[1]
    ei = edge_index.astype(jnp.int32)
    nch = E // (_NW * _CH)
    src2_i = (ei[0] * 2).reshape(_NW, nch, _CH)
    src2b_i = (ei[0] * 2 + 1).reshape(_NW, nch, _CH)
    dst_i = ei[1].reshape(_NW, nch, _CH)
    npad = -(-N // (8 * _NS)) * (8 * _NS)
    zeros_nd = jnp.zeros((npad, _HD), jnp.float32)
    zeros_nc = jnp.zeros((npad, _CW), jnp.float32)
    ones_c = jnp.ones((_CH, _CW), jnp.float32)

    c1 = _sc_count(dst_i, zeros_nc, ones_c)
    xl, xr = _tc_pre(x, W1_l, W1_r, b1_l)
    p1 = _sc_agg(xl.reshape(2 * N, _HD), src2_i, src2b_i, dst_i, zeros_nd)
    hl, hr = _tc_mid(p1, c1, xr, W2_l, W2_r, b2_l)
    p2 = _sc_agg(hl.reshape(2 * N, _HD), src2_i, src2b_i, dst_i, zeros_nd)
    return _tc_post(p2, c1, hr)
